# Initial kernel scaffold; baseline (speedup 1.0000x reference)
import jax
import jax.numpy as jnp
from jax.experimental import pallas as pl


def kernel(x, mask):
    raise NotImplementedError("write your pallas kernel here")


# speedup vs baseline: 12.3982x; 12.3982x over previous
import functools

import jax
import jax.numpy as jnp
import numpy as np
from jax.experimental import pallas as pl
from jax.experimental.pallas import tpu as pltpu

_K = 64
_INT_MIN = np.int32(-2147483648)
_M31 = np.int32(0x7FFFFFFF)


def _to_key(v):
    b = jax.lax.bitcast_convert_type(v, jnp.int32)
    return b ^ ((b >> 31) & _M31)


def _from_key(k):
    return jax.lax.bitcast_convert_type(k ^ ((k >> 31) & _M31), jnp.float32)


def _topk_kernel(x_ref, m_ref, o_ref, keys_ref, *, n_cells, n_feat, k):
    xb = x_ref[0]
    mb = m_ref[0]
    r = xb.shape[0]

    grp = jax.lax.broadcasted_iota(jnp.int32, (1, 128), 1) >> 5
    mexp = jnp.where(
        grp == 0, mb[:, 0:1],
        jnp.where(grp == 1, mb[:, 1:2],
                  jnp.where(grp == 2, mb[:, 2:3], mb[:, 3:4])))

    keys = jnp.where(mexp > 0.5, _INT_MIN, _to_key(xb))
    keys_ref[...] = keys
    n_valid = jnp.float32(n_cells) - jnp.sum(mb)

    def lane_tot(v):
        return (v + pltpu.roll(v, 32, 1) + pltpu.roll(v, 64, 1)
                + pltpu.roll(v, 96, 1))

    def body(i, t):
        cand = t + (jnp.int32(1) << (jnp.int32(31) - i))
        cnt = jnp.sum((keys_ref[...] >= cand).astype(jnp.int32), axis=0,
                      keepdims=True)
        tot = lane_tot(cnt)
        return jnp.where(tot >= k, cand, t)

    t0 = jnp.full((1, 128), _INT_MIN, jnp.int32)
    tstar = jax.lax.fori_loop(0, 32, body, t0)

    keys = keys_ref[...]
    predg = keys > tstar
    cntg = jnp.sum(predg.astype(jnp.int32), axis=0, keepdims=True)
    sumg = jnp.sum(jnp.where(predg, _from_key(keys), 0.0), axis=0,
                   keepdims=True)
    cnt_tot = lane_tot(cntg).astype(jnp.float32)
    sum_tot = lane_tot(sumg)
    tie = jnp.where(tstar == _INT_MIN, 0.0, _from_key(tstar))
    denom = jnp.maximum(jnp.minimum(jnp.float32(k), n_valid), 1.0)
    res = (sum_tot + (jnp.float32(k) - cnt_tot) * tie) / denom
    o_ref[0] = res[:, :n_feat]


def kernel(x, mask):
    b, n, f = x.shape
    assert f == 32 and n % 4 == 0
    r = n // 4
    k = min(_K, n)
    x2 = x.reshape(b, r, 128)
    m4 = mask.reshape(b, r, 4).astype(jnp.float32)

    out = pl.pallas_call(
        functools.partial(_topk_kernel, n_cells=n, n_feat=f, k=k),
        grid=(b,),
        in_specs=[
            pl.BlockSpec((1, r, 128), lambda i: (i, 0, 0)),
            pl.BlockSpec((1, r, 4), lambda i: (i, 0, 0)),
        ],
        out_specs=pl.BlockSpec((1, 1, f), lambda i: (i, 0, 0)),
        out_shape=jax.ShapeDtypeStruct((b, 1, f), jnp.float32),
        scratch_shapes=[pltpu.VMEM((r, 128), jnp.int32)],
    )(x2, m4)
    return out.reshape(b, f)


# speedup vs baseline: 22.8246x; 1.8410x over previous
import functools

import jax
import jax.numpy as jnp
import numpy as np
from jax import lax
from jax.experimental import pallas as pl
from jax.experimental.pallas import tpu as pltpu
from jax.experimental.pallas import tpu_sc as plsc

_B, _N, _F = 32, 32768, 32
_L = 16
_K = 64
_CAP = 3072
_CHUNK = 512
_NCHUNK = _N // _CHUNK
_NG = 64
_GS = _CAP // _NG
_COMPACT_AT = _CAP - _CHUNK
_INT_MIN = np.int32(-2147483648)
_M31 = np.int32(0x7FFFFFFF)


def _key_of(v):
    b = plsc.bitcast(v, jnp.int32)
    return b ^ ((b >> 31) & _M31)


def _val_of(k):
    return plsc.bitcast(k ^ ((k >> 31) & _M31), jnp.float32)


def _lane_idx():
    return lax.broadcasted_iota(jnp.int32, (_L,), 0)


def _xlane_max(v):
    idx = _lane_idx()
    for sh in (8, 4, 2, 1):
        v = jnp.maximum(v, v.at[idx ^ sh].get(mode="promise_in_bounds"))
    return v


def _xlane_sum(v):
    idx = _lane_idx()
    for sh in (8, 4, 2, 1):
        v = v + v.at[idx ^ sh].get(mode="promise_in_bounds")
    return v


def _sc_body(x_hbm, pen_hbm, out_hbm, xv, pv, buf, ov):
    w = lax.axis_index("s") * 2 + lax.axis_index("c")
    iota = lax.broadcasted_iota(jnp.int32, (_L,), 0)
    ninf = jnp.full((_L,), -jnp.inf, jnp.float32)
    pinf = jnp.full((_L,), jnp.inf, jnp.float32)
    zi = jnp.zeros((_L,), jnp.int32)

    def slot_lo(s):
        return buf[pl.ds(s * _F, _L)]

    def slot_hi(s):
        return buf[pl.ds(s * _F + _L, _L)]

    def group_bound(cnt_lo, cnt_hi):
        def g_body(g, acc):
            def s_body(s, m):
                sl = s * _NG + g
                vlo = jnp.where(sl < cnt_lo, slot_lo(sl), ninf)
                vhi = jnp.where(sl < cnt_hi, slot_hi(sl), ninf)
                return (jnp.maximum(m[0], vlo), jnp.maximum(m[1], vhi))
            gm = lax.fori_loop(0, _GS, s_body, (ninf, ninf))
            return (jnp.minimum(acc[0], gm[0]), jnp.minimum(acc[1], gm[1]))
        return lax.fori_loop(0, _NG, g_body, (pinf, pinf))

    def rewrite(cnt_lo, cnt_hi, vlb_lo, vlb_hi, cnt_max, as_keys):
        def s_body(s, c):
            nc_lo, nc_hi, tr_lo, tr_hi = c
            vlo = slot_lo(s)
            vhi = slot_hi(s)
            val_lo = s < cnt_lo
            val_hi = s < cnt_hi
            eq_lo = val_lo & (vlo == vlb_lo) & (tr_lo < _K)
            eq_hi = val_hi & (vhi == vlb_hi) & (tr_hi < _K)
            keep_lo = (val_lo & (vlo > vlb_lo)) | eq_lo
            keep_hi = (val_hi & (vhi > vlb_hi)) | eq_hi
            if as_keys:
                wlo = plsc.bitcast(_key_of(vlo), jnp.float32)
                whi = plsc.bitcast(_key_of(vhi), jnp.float32)
            else:
                wlo, whi = vlo, vhi
            plsc.store_scatter(buf, [(nc_lo * _F) + iota], wlo, mask=keep_lo)
            plsc.store_scatter(buf, [(nc_hi * _F) + iota + _L], whi, mask=keep_hi)
            return (nc_lo + jnp.where(keep_lo, 1, 0),
                    nc_hi + jnp.where(keep_hi, 1, 0),
                    tr_lo + jnp.where(eq_lo, 1, 0),
                    tr_hi + jnp.where(eq_hi, 1, 0))
        nc_lo, nc_hi, _, _ = lax.fori_loop(0, cnt_max, s_body,
                                           (zi, zi, zi, zi))
        return nc_lo, nc_hi

    cf_cap = jnp.full((_L,), (_CAP - 1) * _F, jnp.int32)

    def chunk_body(ch, carry):
        tau_lo, tau_hi, cf_lo, cf_hi, nval = carry
        pltpu.sync_copy(x_hbm.at[w, pl.ds(ch * _CHUNK * _F, _CHUNK * _F)], xv)
        pltpu.sync_copy(pen_hbm.at[w, pl.ds(ch * _CHUNK, _CHUNK)], pv)

        def cell16(q, c):
            cfl, cfh, nv = c
            pvec = pv[pl.ds(q * _L, _L)]
            nv = nv + jnp.where(pvec > 0.0, 1, 0)
            for u in range(_L):
                cell = q * _L + u
                p = pvec[u]
                vlo = jnp.minimum(xv[pl.ds(cell * _F, _L)], p)
                vhi = jnp.minimum(xv[pl.ds(cell * _F + _L, _L)], p)
                klo = vlo > tau_lo
                khi = vhi > tau_hi
                plsc.store_scatter(buf, [cfl + iota], vlo, mask=klo)
                plsc.store_scatter(buf, [cfh + iota + _L], vhi, mask=khi)
                cfl = jnp.minimum(cfl + jnp.where(klo, _F, 0), cf_cap)
                cfh = jnp.minimum(cfh + jnp.where(khi, _F, 0), cf_cap)
            return (cfl, cfh, nv)

        cf_lo, cf_hi, nval = lax.fori_loop(0, _CHUNK // _L, cell16,
                                           (cf_lo, cf_hi, nval))
        cnt_lo = lax.shift_right_arithmetic(cf_lo, 5)
        cnt_hi = lax.shift_right_arithmetic(cf_hi, 5)
        cnt_max = _xlane_max(jnp.maximum(cnt_lo, cnt_hi))[0]

        def do_compact(op):
            tl, th, clo, chi, cm = op
            vlb_lo, vlb_hi = group_bound(clo, chi)
            ncl, nch = rewrite(clo, chi, vlb_lo, vlb_hi, cm, False)
            return (jnp.maximum(tl, vlb_lo), jnp.maximum(th, vlb_hi),
                    ncl, nch, cm)

        tau_lo, tau_hi, cnt_lo, cnt_hi, _ = lax.cond(
            cnt_max > _COMPACT_AT, do_compact, lambda op: op,
            (tau_lo, tau_hi, cnt_lo, cnt_hi, cnt_max))
        return (tau_lo, tau_hi, cnt_lo * _F, cnt_hi * _F, nval)

    tau0 = jnp.full((_L,), -jnp.inf, jnp.float32)
    tau_lo, tau_hi, cf_lo, cf_hi, nval = lax.fori_loop(
        0, _NCHUNK, chunk_body, (tau0, tau0, zi, zi, zi))

    cnt_lo = lax.shift_right_arithmetic(cf_lo, 5)
    cnt_hi = lax.shift_right_arithmetic(cf_hi, 5)
    cnt_max = _xlane_max(jnp.maximum(cnt_lo, cnt_hi))[0]
    vlb_lo, vlb_hi = group_bound(cnt_lo, cnt_hi)
    nc_lo, nc_hi = rewrite(cnt_lo, cnt_hi, vlb_lo, vlb_hi, cnt_max, True)
    nc_max = _xlane_max(jnp.maximum(nc_lo, nc_hi))[0]

    kmin_f = plsc.bitcast(jnp.full((_L,), _INT_MIN, jnp.int32), jnp.float32)

    def clear_body(s, _):
        plsc.store_scatter(buf, [(s * _F) + iota], kmin_f, mask=s >= nc_lo)
        plsc.store_scatter(buf, [(s * _F) + iota + _L], kmin_f, mask=s >= nc_hi)
        return 0

    lax.fori_loop(0, nc_max, clear_body, 0)

    def bit_body(i, t):
        t_lo, t_hi = t
        bit = jnp.int32(1) << (jnp.int32(31) - i)
        cand_lo = t_lo + bit
        cand_hi = t_hi + bit

        def s_body(s, c):
            k_lo = plsc.bitcast(slot_lo(s), jnp.int32)
            k_hi = plsc.bitcast(slot_hi(s), jnp.int32)
            return (c[0] + jnp.where(k_lo >= cand_lo, 1, 0),
                    c[1] + jnp.where(k_hi >= cand_hi, 1, 0))

        c_lo, c_hi = lax.fori_loop(0, nc_max, s_body, (zi, zi))
        return (jnp.where(c_lo >= _K, cand_lo, t_lo),
                jnp.where(c_hi >= _K, cand_hi, t_hi))

    tmin = jnp.full((_L,), _INT_MIN, jnp.int32)
    t_lo, t_hi = lax.fori_loop(0, 32, bit_body, (tmin, tmin))

    def stat_body(s, c):
        sg_lo, sg_hi, cg_lo, cg_hi = c
        k_lo = plsc.bitcast(slot_lo(s), jnp.int32)
        k_hi = plsc.bitcast(slot_hi(s), jnp.int32)
        g_lo = k_lo > t_lo
        g_hi = k_hi > t_hi
        return (sg_lo + jnp.where(g_lo, _val_of(k_lo), 0.0),
                sg_hi + jnp.where(g_hi, _val_of(k_hi), 0.0),
                cg_lo + jnp.where(g_lo, 1, 0),
                cg_hi + jnp.where(g_hi, 1, 0))

    zf = jnp.zeros((_L,), jnp.float32)
    sg_lo, sg_hi, cg_lo, cg_hi = lax.fori_loop(0, nc_max, stat_body,
                                               (zf, zf, zi, zi))

    tie_lo = jnp.where(t_lo == _INT_MIN, 0.0, _val_of(t_lo))
    tie_hi = jnp.where(t_hi == _INT_MIN, 0.0, _val_of(t_hi))
    kf = jnp.float32(_K)
    denom = jnp.maximum(jnp.minimum(kf, _xlane_sum(nval).astype(jnp.float32)), 1.0)
    res_lo = (sg_lo + (kf - cg_lo.astype(jnp.float32)) * tie_lo) / denom
    res_hi = (sg_hi + (kf - cg_hi.astype(jnp.float32)) * tie_hi) / denom
    ov[pl.ds(0, _L)] = res_lo
    ov[pl.ds(_L, _L)] = res_hi
    pltpu.sync_copy(ov, out_hbm.at[w])


@jax.jit
def _sc_call(x, pen):
    mesh = plsc.VectorSubcoreMesh(core_axis_name="c", subcore_axis_name="s")
    return pl.kernel(
        _sc_body,
        mesh=mesh,
        compiler_params=pltpu.CompilerParams(needs_layout_passes=False),
        out_type=jax.ShapeDtypeStruct((_B, _F), jnp.float32),
        scratch_types=[
            pltpu.VMEM((_CHUNK * _F,), jnp.float32),
            pltpu.VMEM((_CHUNK,), jnp.float32),
            pltpu.VMEM((_CAP * _F,), jnp.float32),
            pltpu.VMEM((_F,), jnp.float32),
        ],
    )(x, pen)


def kernel(x, mask):
    b, n, f = x.shape
    assert (b, n, f) == (_B, _N, _F)
    pen = jnp.where(mask, -jnp.inf, jnp.inf).astype(jnp.float32)
    return _sc_call(x.reshape(_B, _N * _F), pen)


# speedup vs baseline: 24.9154x; 1.0916x over previous
import functools

import jax
import jax.numpy as jnp
import numpy as np
from jax import lax
from jax.experimental import pallas as pl
from jax.experimental.pallas import tpu as pltpu
from jax.experimental.pallas import tpu_sc as plsc

_B, _N, _F = 32, 32768, 32
_L = 16
_K = 64
_CAP = 1792
_CHUNK = 512
_NCHUNK = _N // _CHUNK
_NG = 64
_GS = _CAP // _NG
_COMPACT_AT = _CAP - _CHUNK - 20
_INT_MIN = np.int32(-2147483648)
_M31 = np.int32(0x7FFFFFFF)


def _key_of(v):
    b = plsc.bitcast(v, jnp.int32)
    return b ^ ((b >> 31) & _M31)


def _val_of(k):
    return plsc.bitcast(k ^ ((k >> 31) & _M31), jnp.float32)


def _lane_idx():
    return lax.broadcasted_iota(jnp.int32, (_L,), 0)


def _xlane_max(v):
    idx = _lane_idx()
    for sh in (8, 4, 2, 1):
        v = jnp.maximum(v, v.at[idx ^ sh].get(mode="promise_in_bounds"))
    return v


def _xlane_sum(v):
    idx = _lane_idx()
    for sh in (8, 4, 2, 1):
        v = v + v.at[idx ^ sh].get(mode="promise_in_bounds")
    return v


def _sc_body(x_hbm, pen_hbm, out_hbm, xv, pv, buf, ov):
    w = lax.axis_index("s") * 2 + lax.axis_index("c")
    iota = lax.broadcasted_iota(jnp.int32, (_L,), 0)
    ninf = jnp.full((_L,), -jnp.inf, jnp.float32)
    pinf = jnp.full((_L,), jnp.inf, jnp.float32)
    zi = jnp.zeros((_L,), jnp.int32)

    def slot_lo(s):
        return buf[pl.ds(s * _F, _L)]

    def slot_hi(s):
        return buf[pl.ds(s * _F + _L, _L)]

    def group_bound(cnt_lo, cnt_hi):
        def g_body(g, acc):
            def s_body(s, m):
                sl = s * _NG + g
                vlo = jnp.where(sl < cnt_lo, slot_lo(sl), ninf)
                vhi = jnp.where(sl < cnt_hi, slot_hi(sl), ninf)
                return (jnp.maximum(m[0], vlo), jnp.maximum(m[1], vhi))
            gm = lax.fori_loop(0, _GS, s_body, (ninf, ninf))
            return (jnp.minimum(acc[0], gm[0]), jnp.minimum(acc[1], gm[1]))
        return lax.fori_loop(0, _NG, g_body, (pinf, pinf))

    def rewrite(cnt_lo, cnt_hi, vlb_lo, vlb_hi, cnt_max, as_keys):
        def s_body(s, c):
            nc_lo, nc_hi, tr_lo, tr_hi = c
            vlo = slot_lo(s)
            vhi = slot_hi(s)
            val_lo = s < cnt_lo
            val_hi = s < cnt_hi
            eq_lo = val_lo & (vlo == vlb_lo) & (tr_lo < _K)
            eq_hi = val_hi & (vhi == vlb_hi) & (tr_hi < _K)
            keep_lo = (val_lo & (vlo > vlb_lo)) | eq_lo
            keep_hi = (val_hi & (vhi > vlb_hi)) | eq_hi
            if as_keys:
                wlo = plsc.bitcast(_key_of(vlo), jnp.float32)
                whi = plsc.bitcast(_key_of(vhi), jnp.float32)
            else:
                wlo, whi = vlo, vhi
            plsc.store_scatter(buf, [(nc_lo * _F) + iota], wlo, mask=keep_lo)
            plsc.store_scatter(buf, [(nc_hi * _F) + iota + _L], whi, mask=keep_hi)
            return (nc_lo + jnp.where(keep_lo, 1, 0),
                    nc_hi + jnp.where(keep_hi, 1, 0),
                    tr_lo + jnp.where(eq_lo, 1, 0),
                    tr_hi + jnp.where(eq_hi, 1, 0))
        nc_lo, nc_hi, _, _ = lax.fori_loop(0, cnt_max, s_body,
                                           (zi, zi, zi, zi))
        return nc_lo, nc_hi

    cf_cap = jnp.full((_L,), (_CAP - 17) * _F, jnp.int32)

    def chunk_body(ch, carry):
        tau_lo, tau_hi, cf_lo, cf_hi, nval = carry
        pltpu.sync_copy(x_hbm.at[w, pl.ds(ch * _CHUNK, _CHUNK)], xv)
        pltpu.sync_copy(pen_hbm.at[w, pl.ds(ch * _CHUNK, _CHUNK)], pv)

        def cell16(q, c):
            cfl, cfh, nv = c
            pvec = pv[pl.ds(q * _L, _L)]
            nv = nv + jnp.where(pvec > 0.0, 1, 0)
            for u in range(_L):
                cell = q * _L + u
                p = pvec[u]
                vlo = jnp.minimum(xv[cell, 0:_L], p)
                vhi = jnp.minimum(xv[cell, _L:2 * _L], p)
                klo = vlo > tau_lo
                khi = vhi > tau_hi
                plsc.store_scatter(buf, [cfl + iota], vlo, mask=klo)
                plsc.store_scatter(buf, [cfh + iota + _L], vhi, mask=khi)
                cfl = cfl + jnp.where(klo, _F, 0)
                cfh = cfh + jnp.where(khi, _F, 0)
            return (jnp.minimum(cfl, cf_cap), jnp.minimum(cfh, cf_cap), nv)

        cf_lo, cf_hi, nval = lax.fori_loop(0, _CHUNK // _L, cell16,
                                           (cf_lo, cf_hi, nval))
        cnt_lo = lax.shift_right_arithmetic(cf_lo, 5)
        cnt_hi = lax.shift_right_arithmetic(cf_hi, 5)
        cnt_max = _xlane_max(jnp.maximum(cnt_lo, cnt_hi))[0]

        def do_compact(op):
            tl, th, clo, chi, cm = op
            vlb_lo, vlb_hi = group_bound(clo, chi)
            ncl, nch = rewrite(clo, chi, vlb_lo, vlb_hi, cm, False)
            return (jnp.maximum(tl, vlb_lo), jnp.maximum(th, vlb_hi),
                    ncl, nch, cm)

        tau_lo, tau_hi, cnt_lo, cnt_hi, _ = lax.cond(
            cnt_max > _COMPACT_AT, do_compact, lambda op: op,
            (tau_lo, tau_hi, cnt_lo, cnt_hi, cnt_max))
        return (tau_lo, tau_hi, cnt_lo * _F, cnt_hi * _F, nval)

    tau0 = jnp.full((_L,), -jnp.inf, jnp.float32)
    tau_lo, tau_hi, cf_lo, cf_hi, nval = lax.fori_loop(
        0, _NCHUNK, chunk_body, (tau0, tau0, zi, zi, zi))

    cnt_lo = lax.shift_right_arithmetic(cf_lo, 5)
    cnt_hi = lax.shift_right_arithmetic(cf_hi, 5)
    cnt_max = _xlane_max(jnp.maximum(cnt_lo, cnt_hi))[0]
    vlb_lo, vlb_hi = group_bound(cnt_lo, cnt_hi)
    nc_lo, nc_hi = rewrite(cnt_lo, cnt_hi, vlb_lo, vlb_hi, cnt_max, True)
    nc_max = _xlane_max(jnp.maximum(nc_lo, nc_hi))[0]

    kmin_f = plsc.bitcast(jnp.full((_L,), _INT_MIN, jnp.int32), jnp.float32)

    def clear_body(s, _):
        plsc.store_scatter(buf, [(s * _F) + iota], kmin_f, mask=s >= nc_lo)
        plsc.store_scatter(buf, [(s * _F) + iota + _L], kmin_f, mask=s >= nc_hi)
        return 0

    lax.fori_loop(0, nc_max, clear_body, 0)

    def bit_body(i, t):
        t_lo, t_hi = t
        bit = jnp.int32(1) << (jnp.int32(31) - i)
        cand_lo = t_lo + bit
        cand_hi = t_hi + bit

        def s_body(s, c):
            k_lo = plsc.bitcast(slot_lo(s), jnp.int32)
            k_hi = plsc.bitcast(slot_hi(s), jnp.int32)
            return (c[0] + jnp.where(k_lo >= cand_lo, 1, 0),
                    c[1] + jnp.where(k_hi >= cand_hi, 1, 0))

        c_lo, c_hi = lax.fori_loop(0, nc_max, s_body, (zi, zi))
        return (jnp.where(c_lo >= _K, cand_lo, t_lo),
                jnp.where(c_hi >= _K, cand_hi, t_hi))

    tmin = jnp.full((_L,), _INT_MIN, jnp.int32)
    t_lo, t_hi = lax.fori_loop(0, 32, bit_body, (tmin, tmin))

    def stat_body(s, c):
        sg_lo, sg_hi, cg_lo, cg_hi = c
        k_lo = plsc.bitcast(slot_lo(s), jnp.int32)
        k_hi = plsc.bitcast(slot_hi(s), jnp.int32)
        g_lo = k_lo > t_lo
        g_hi = k_hi > t_hi
        return (sg_lo + jnp.where(g_lo, _val_of(k_lo), 0.0),
                sg_hi + jnp.where(g_hi, _val_of(k_hi), 0.0),
                cg_lo + jnp.where(g_lo, 1, 0),
                cg_hi + jnp.where(g_hi, 1, 0))

    zf = jnp.zeros((_L,), jnp.float32)
    sg_lo, sg_hi, cg_lo, cg_hi = lax.fori_loop(0, nc_max, stat_body,
                                               (zf, zf, zi, zi))

    tie_lo = jnp.where(t_lo == _INT_MIN, 0.0, _val_of(t_lo))
    tie_hi = jnp.where(t_hi == _INT_MIN, 0.0, _val_of(t_hi))
    kf = jnp.float32(_K)
    denom = jnp.maximum(jnp.minimum(kf, _xlane_sum(nval).astype(jnp.float32)), 1.0)
    res_lo = (sg_lo + (kf - cg_lo.astype(jnp.float32)) * tie_lo) / denom
    res_hi = (sg_hi + (kf - cg_hi.astype(jnp.float32)) * tie_hi) / denom
    ov[pl.ds(0, _L)] = res_lo
    ov[pl.ds(_L, _L)] = res_hi
    pltpu.sync_copy(ov, out_hbm.at[w])


@jax.jit
def _sc_call(x, pen):
    mesh = plsc.VectorSubcoreMesh(core_axis_name="c", subcore_axis_name="s")
    return pl.kernel(
        _sc_body,
        mesh=mesh,
        compiler_params=pltpu.CompilerParams(needs_layout_passes=False),
        out_type=jax.ShapeDtypeStruct((_B, _F), jnp.float32),
        scratch_types=[
            pltpu.VMEM((_CHUNK, _F), jnp.float32),
            pltpu.VMEM((_CHUNK,), jnp.float32),
            pltpu.VMEM((_CAP * _F,), jnp.float32),
            pltpu.VMEM((_F,), jnp.float32),
        ],
    )(x, pen)


def kernel(x, mask):
    b, n, f = x.shape
    assert (b, n, f) == (_B, _N, _F)
    pen = jnp.where(mask, -jnp.inf, jnp.inf).astype(jnp.float32)
    return _sc_call(x, pen)


# speedup vs baseline: 24.9291x; 1.0005x over previous
import functools

import jax
import jax.numpy as jnp
import numpy as np
from jax import lax
from jax.experimental import pallas as pl
from jax.experimental.pallas import tpu as pltpu
from jax.experimental.pallas import tpu_sc as plsc

_B, _N, _F = 32, 32768, 32
_L = 16
_K = 64
_CAP = 1792
_CHUNK = 512
_NCHUNK = _N // _CHUNK
_NG = 64
_GS = _CAP // _NG
_COMPACT_AT = _CAP - _CHUNK - 20
_INT_MIN = np.int32(-2147483648)
_M31 = np.int32(0x7FFFFFFF)


def _key_of(v):
    b = plsc.bitcast(v, jnp.int32)
    return b ^ ((b >> 31) & _M31)


def _val_of(k):
    return plsc.bitcast(k ^ ((k >> 31) & _M31), jnp.float32)


def _lane_idx():
    return lax.broadcasted_iota(jnp.int32, (_L,), 0)


def _xlane_max(v):
    idx = _lane_idx()
    for sh in (8, 4, 2, 1):
        v = jnp.maximum(v, v.at[idx ^ sh].get(mode="promise_in_bounds"))
    return v


def _xlane_sum(v):
    idx = _lane_idx()
    for sh in (8, 4, 2, 1):
        v = v + v.at[idx ^ sh].get(mode="promise_in_bounds")
    return v


def _sc_body(x_hbm, pen_hbm, out_hbm, xv, pv, buf, ov):
    w = lax.axis_index("s") * 2 + lax.axis_index("c")
    iota = lax.broadcasted_iota(jnp.int32, (_L,), 0)
    ninf = jnp.full((_L,), -jnp.inf, jnp.float32)
    pinf = jnp.full((_L,), jnp.inf, jnp.float32)
    zi = jnp.zeros((_L,), jnp.int32)

    def slot_lo(s):
        return buf[pl.ds(s * _F, _L)]

    def slot_hi(s):
        return buf[pl.ds(s * _F + _L, _L)]

    def group_bound(cnt_lo, cnt_hi):
        def g_body(g, acc):
            def s_body(s, m):
                sl = s * _NG + g
                vlo = jnp.where(sl < cnt_lo, slot_lo(sl), ninf)
                vhi = jnp.where(sl < cnt_hi, slot_hi(sl), ninf)
                return (jnp.maximum(m[0], vlo), jnp.maximum(m[1], vhi))
            gm = lax.fori_loop(0, _GS, s_body, (ninf, ninf))
            return (jnp.minimum(acc[0], gm[0]), jnp.minimum(acc[1], gm[1]))
        return lax.fori_loop(0, _NG, g_body, (pinf, pinf))

    def rewrite(cnt_lo, cnt_hi, vlb_lo, vlb_hi, cnt_max, as_keys):
        def s_body(s, c):
            nc_lo, nc_hi, tr_lo, tr_hi = c
            vlo = slot_lo(s)
            vhi = slot_hi(s)
            val_lo = s < cnt_lo
            val_hi = s < cnt_hi
            eq_lo = val_lo & (vlo == vlb_lo) & (tr_lo < _K)
            eq_hi = val_hi & (vhi == vlb_hi) & (tr_hi < _K)
            keep_lo = (val_lo & (vlo > vlb_lo)) | eq_lo
            keep_hi = (val_hi & (vhi > vlb_hi)) | eq_hi
            if as_keys:
                wlo = plsc.bitcast(_key_of(vlo), jnp.float32)
                whi = plsc.bitcast(_key_of(vhi), jnp.float32)
            else:
                wlo, whi = vlo, vhi
            plsc.store_scatter(buf, [(nc_lo * _F) + iota], wlo, mask=keep_lo)
            plsc.store_scatter(buf, [(nc_hi * _F) + iota + _L], whi, mask=keep_hi)
            return (nc_lo + jnp.where(keep_lo, 1, 0),
                    nc_hi + jnp.where(keep_hi, 1, 0),
                    tr_lo + jnp.where(eq_lo, 1, 0),
                    tr_hi + jnp.where(eq_hi, 1, 0))
        nc_lo, nc_hi, _, _ = lax.fori_loop(0, cnt_max, s_body,
                                           (zi, zi, zi, zi))
        return nc_lo, nc_hi

    cf_cap = jnp.full((_L,), (_CAP - 17) * _F, jnp.int32)

    def chunk_body(ch, carry):
        tau_lo, tau_hi, cf_lo, cf_hi, nval = carry
        pltpu.sync_copy(x_hbm.at[w, pl.ds(ch * _CHUNK, _CHUNK)], xv)
        pltpu.sync_copy(pen_hbm.at[w, pl.ds(ch * _CHUNK, _CHUNK)], pv)

        @plsc.parallel_loop(0, _CHUNK // _L, carry=(cf_lo, cf_hi, nval))
        def cell16(q, c):
            cfl, cfh, nv = c
            pvec = pv[pl.ds(q * _L, _L)]
            nv = nv + jnp.where(pvec > 0.0, 1, 0)
            for u in range(_L):
                cell = q * _L + u
                p = pvec[u]
                vlo = jnp.minimum(xv[cell, 0:_L], p)
                vhi = jnp.minimum(xv[cell, _L:2 * _L], p)
                klo = vlo > tau_lo
                khi = vhi > tau_hi
                plsc.store_scatter(buf, [cfl + iota], vlo, mask=klo)
                plsc.store_scatter(buf, [cfh + iota + _L], vhi, mask=khi)
                cfl = cfl + jnp.where(klo, _F, 0)
                cfh = cfh + jnp.where(khi, _F, 0)
            return (jnp.minimum(cfl, cf_cap), jnp.minimum(cfh, cf_cap), nv)

        cf_lo, cf_hi, nval = cell16
        cnt_lo = lax.shift_right_arithmetic(cf_lo, 5)
        cnt_hi = lax.shift_right_arithmetic(cf_hi, 5)
        cnt_max = _xlane_max(jnp.maximum(cnt_lo, cnt_hi))[0]

        def do_compact(op):
            tl, th, clo, chi, cm = op
            vlb_lo, vlb_hi = group_bound(clo, chi)
            ncl, nch = rewrite(clo, chi, vlb_lo, vlb_hi, cm, False)
            return (jnp.maximum(tl, vlb_lo), jnp.maximum(th, vlb_hi),
                    ncl, nch, cm)

        tau_lo, tau_hi, cnt_lo, cnt_hi, _ = lax.cond(
            cnt_max > _COMPACT_AT, do_compact, lambda op: op,
            (tau_lo, tau_hi, cnt_lo, cnt_hi, cnt_max))
        return (tau_lo, tau_hi, cnt_lo * _F, cnt_hi * _F, nval)

    tau0 = jnp.full((_L,), -jnp.inf, jnp.float32)
    tau_lo, tau_hi, cf_lo, cf_hi, nval = lax.fori_loop(
        0, _NCHUNK, chunk_body, (tau0, tau0, zi, zi, zi))

    cnt_lo = lax.shift_right_arithmetic(cf_lo, 5)
    cnt_hi = lax.shift_right_arithmetic(cf_hi, 5)
    cnt_max = _xlane_max(jnp.maximum(cnt_lo, cnt_hi))[0]
    vlb_lo, vlb_hi = group_bound(cnt_lo, cnt_hi)
    nc_lo, nc_hi = rewrite(cnt_lo, cnt_hi, vlb_lo, vlb_hi, cnt_max, True)
    nc_max = _xlane_max(jnp.maximum(nc_lo, nc_hi))[0]

    kmin_f = plsc.bitcast(jnp.full((_L,), _INT_MIN, jnp.int32), jnp.float32)

    def clear_body(s, _):
        plsc.store_scatter(buf, [(s * _F) + iota], kmin_f, mask=s >= nc_lo)
        plsc.store_scatter(buf, [(s * _F) + iota + _L], kmin_f, mask=s >= nc_hi)
        return 0

    lax.fori_loop(0, nc_max, clear_body, 0)

    def bit_body(i, t):
        t_lo, t_hi = t
        bit = jnp.int32(1) << (jnp.int32(31) - i)
        cand_lo = t_lo + bit
        cand_hi = t_hi + bit

        def s_body(s, c):
            k_lo = plsc.bitcast(slot_lo(s), jnp.int32)
            k_hi = plsc.bitcast(slot_hi(s), jnp.int32)
            return (c[0] + jnp.where(k_lo >= cand_lo, 1, 0),
                    c[1] + jnp.where(k_hi >= cand_hi, 1, 0))

        c_lo, c_hi = lax.fori_loop(0, nc_max, s_body, (zi, zi))
        return (jnp.where(c_lo >= _K, cand_lo, t_lo),
                jnp.where(c_hi >= _K, cand_hi, t_hi))

    tmin = jnp.full((_L,), _INT_MIN, jnp.int32)
    t_lo, t_hi = lax.fori_loop(0, 32, bit_body, (tmin, tmin))

    def stat_body(s, c):
        sg_lo, sg_hi, cg_lo, cg_hi = c
        k_lo = plsc.bitcast(slot_lo(s), jnp.int32)
        k_hi = plsc.bitcast(slot_hi(s), jnp.int32)
        g_lo = k_lo > t_lo
        g_hi = k_hi > t_hi
        return (sg_lo + jnp.where(g_lo, _val_of(k_lo), 0.0),
                sg_hi + jnp.where(g_hi, _val_of(k_hi), 0.0),
                cg_lo + jnp.where(g_lo, 1, 0),
                cg_hi + jnp.where(g_hi, 1, 0))

    zf = jnp.zeros((_L,), jnp.float32)
    sg_lo, sg_hi, cg_lo, cg_hi = lax.fori_loop(0, nc_max, stat_body,
                                               (zf, zf, zi, zi))

    tie_lo = jnp.where(t_lo == _INT_MIN, 0.0, _val_of(t_lo))
    tie_hi = jnp.where(t_hi == _INT_MIN, 0.0, _val_of(t_hi))
    kf = jnp.float32(_K)
    denom = jnp.maximum(jnp.minimum(kf, _xlane_sum(nval).astype(jnp.float32)), 1.0)
    res_lo = (sg_lo + (kf - cg_lo.astype(jnp.float32)) * tie_lo) / denom
    res_hi = (sg_hi + (kf - cg_hi.astype(jnp.float32)) * tie_hi) / denom
    ov[pl.ds(0, _L)] = res_lo
    ov[pl.ds(_L, _L)] = res_hi
    pltpu.sync_copy(ov, out_hbm.at[w])


@jax.jit
def _sc_call(x, pen):
    mesh = plsc.VectorSubcoreMesh(core_axis_name="c", subcore_axis_name="s")
    return pl.kernel(
        _sc_body,
        mesh=mesh,
        compiler_params=pltpu.CompilerParams(needs_layout_passes=False),
        out_type=jax.ShapeDtypeStruct((_B, _F), jnp.float32),
        scratch_types=[
            pltpu.VMEM((_CHUNK, _F), jnp.float32),
            pltpu.VMEM((_CHUNK,), jnp.float32),
            pltpu.VMEM((_CAP * _F,), jnp.float32),
            pltpu.VMEM((_F,), jnp.float32),
        ],
    )(x, pen)


def kernel(x, mask):
    b, n, f = x.shape
    assert (b, n, f) == (_B, _N, _F)
    pen = jnp.where(mask, -jnp.inf, jnp.inf).astype(jnp.float32)
    return _sc_call(x, pen)


# speedup vs baseline: 31.9588x; 1.2820x over previous
import functools

import jax
import jax.numpy as jnp
import numpy as np
from jax import lax
from jax.experimental import pallas as pl
from jax.experimental.pallas import tpu as pltpu
from jax.experimental.pallas import tpu_sc as plsc

_B, _N, _F = 32, 32768, 32
_L = 16
_K = 64
_CAP = 1792
_CHUNK = 256
_NCHUNK = _N // _CHUNK
_NG = 64
_GS = _CAP // _NG
_COMPACT_AT = _CAP - _CHUNK - 20
_INT_MIN = np.int32(-2147483648)
_M31 = np.int32(0x7FFFFFFF)


def _key_of(v):
    b = plsc.bitcast(v, jnp.int32)
    return b ^ ((b >> 31) & _M31)


def _val_of(k):
    return plsc.bitcast(k ^ ((k >> 31) & _M31), jnp.float32)


def _lane_idx():
    return lax.broadcasted_iota(jnp.int32, (_L,), 0)


def _xlane_max(v):
    idx = _lane_idx()
    for sh in (8, 4, 2, 1):
        v = jnp.maximum(v, v.at[idx ^ sh].get(mode="promise_in_bounds"))
    return v


def _xlane_sum(v):
    idx = _lane_idx()
    for sh in (8, 4, 2, 1):
        v = v + v.at[idx ^ sh].get(mode="promise_in_bounds")
    return v


def _sc_body(x_hbm, pen_hbm, out_hbm, xv0, xv1, pv0, pv1, buf, ov,
             sx0, sx1, sp0, sp1):
    w = lax.axis_index("s") * 2 + lax.axis_index("c")
    iota = lax.broadcasted_iota(jnp.int32, (_L,), 0)
    ninf = jnp.full((_L,), -jnp.inf, jnp.float32)
    pinf = jnp.full((_L,), jnp.inf, jnp.float32)
    zi = jnp.zeros((_L,), jnp.int32)

    def slot_lo(s):
        return buf[pl.ds(s * _F, _L)]

    def slot_hi(s):
        return buf[pl.ds(s * _F + _L, _L)]

    def group_bound(cnt_lo, cnt_hi):
        def g_body(g, acc):
            def s_body(s, m):
                sl = s * _NG + g
                vlo = jnp.where(sl < cnt_lo, slot_lo(sl), ninf)
                vhi = jnp.where(sl < cnt_hi, slot_hi(sl), ninf)
                return (jnp.maximum(m[0], vlo), jnp.maximum(m[1], vhi))
            gm = lax.fori_loop(0, _GS, s_body, (ninf, ninf))
            return (jnp.minimum(acc[0], gm[0]), jnp.minimum(acc[1], gm[1]))
        return lax.fori_loop(0, _NG, g_body, (pinf, pinf))

    def rewrite(cnt_lo, cnt_hi, vlb_lo, vlb_hi, cnt_max, as_keys):
        def s_body(s, c):
            nc_lo, nc_hi, tr_lo, tr_hi = c
            vlo = slot_lo(s)
            vhi = slot_hi(s)
            val_lo = s < cnt_lo
            val_hi = s < cnt_hi
            eq_lo = val_lo & (vlo == vlb_lo) & (tr_lo < _K)
            eq_hi = val_hi & (vhi == vlb_hi) & (tr_hi < _K)
            keep_lo = (val_lo & (vlo > vlb_lo)) | eq_lo
            keep_hi = (val_hi & (vhi > vlb_hi)) | eq_hi
            if as_keys:
                wlo = plsc.bitcast(_key_of(vlo), jnp.float32)
                whi = plsc.bitcast(_key_of(vhi), jnp.float32)
            else:
                wlo, whi = vlo, vhi
            plsc.store_scatter(buf, [(nc_lo * _F) + iota], wlo, mask=keep_lo)
            plsc.store_scatter(buf, [(nc_hi * _F) + iota + _L], whi, mask=keep_hi)
            return (nc_lo + jnp.where(keep_lo, 1, 0),
                    nc_hi + jnp.where(keep_hi, 1, 0),
                    tr_lo + jnp.where(eq_lo, 1, 0),
                    tr_hi + jnp.where(eq_hi, 1, 0))
        nc_lo, nc_hi, _, _ = lax.fori_loop(0, cnt_max, s_body,
                                           (zi, zi, zi, zi))
        return nc_lo, nc_hi

    cf_cap = jnp.full((_L,), (_CAP - 17) * _F, jnp.int32)

    xvs, pvs, sxs, sps = (xv0, xv1), (pv0, pv1), (sx0, sx1), (sp0, sp1)

    def issue(ch, b):
        pltpu.async_copy(x_hbm.at[w, pl.ds(ch * _CHUNK, _CHUNK)], xvs[b],
                         sxs[b])
        pltpu.async_copy(pen_hbm.at[w, pl.ds(ch * _CHUNK, _CHUNK)], pvs[b],
                         sps[b])

    issue(0, 0)
    issue(1, 1)

    def chunk_body(ch, carry, b):
        xv, pv = xvs[b], pvs[b]
        tau_lo, tau_hi, cf_lo, cf_hi, nval = carry
        pltpu.make_async_copy(x_hbm.at[w, pl.ds(0, _CHUNK)], xv,
                              sxs[b]).wait()
        pltpu.make_async_copy(pen_hbm.at[w, pl.ds(0, _CHUNK)], pv,
                              sps[b]).wait()

        @plsc.parallel_loop(0, _CHUNK // _L, carry=(cf_lo, cf_hi, nval))
        def cell16(q, c):
            cfl, cfh, nv = c
            pvec = pv[pl.ds(q * _L, _L)]
            nv = nv + jnp.where(pvec > 0.0, 1, 0)
            for u in range(_L):
                cell = q * _L + u
                p = pvec[u]
                vlo = jnp.minimum(xv[cell, 0:_L], p)
                vhi = jnp.minimum(xv[cell, _L:2 * _L], p)
                klo = vlo > tau_lo
                khi = vhi > tau_hi
                plsc.store_scatter(buf, [cfl + iota], vlo, mask=klo)
                plsc.store_scatter(buf, [cfh + iota + _L], vhi, mask=khi)
                cfl = cfl + jnp.where(klo, _F, 0)
                cfh = cfh + jnp.where(khi, _F, 0)
            return (jnp.minimum(cfl, cf_cap), jnp.minimum(cfh, cf_cap), nv)

        cf_lo, cf_hi, nval = cell16
        pl.when(ch + 2 < _NCHUNK)(lambda: issue(ch + 2, b))
        cnt_lo = lax.shift_right_arithmetic(cf_lo, 5)
        cnt_hi = lax.shift_right_arithmetic(cf_hi, 5)
        cnt_max = _xlane_max(jnp.maximum(cnt_lo, cnt_hi))[0]

        def do_compact(op):
            tl, th, clo, chi, cm = op
            vlb_lo, vlb_hi = group_bound(clo, chi)
            ncl, nch = rewrite(clo, chi, vlb_lo, vlb_hi, cm, False)
            return (jnp.maximum(tl, vlb_lo), jnp.maximum(th, vlb_hi),
                    ncl, nch, cm)

        tau_lo, tau_hi, cnt_lo, cnt_hi, _ = lax.cond(
            cnt_max > _COMPACT_AT, do_compact, lambda op: op,
            (tau_lo, tau_hi, cnt_lo, cnt_hi, cnt_max))
        return (tau_lo, tau_hi, cnt_lo * _F, cnt_hi * _F, nval)

    def pair_body(pr, carry):
        carry = chunk_body(pr * 2, carry, 0)
        return chunk_body(pr * 2 + 1, carry, 1)

    tau0 = jnp.full((_L,), -jnp.inf, jnp.float32)
    tau_lo, tau_hi, cf_lo, cf_hi, nval = lax.fori_loop(
        0, _NCHUNK // 2, pair_body, (tau0, tau0, zi, zi, zi))

    cnt_lo = lax.shift_right_arithmetic(cf_lo, 5)
    cnt_hi = lax.shift_right_arithmetic(cf_hi, 5)
    cnt_max = _xlane_max(jnp.maximum(cnt_lo, cnt_hi))[0]
    vlb_lo, vlb_hi = group_bound(cnt_lo, cnt_hi)
    nc_lo, nc_hi = rewrite(cnt_lo, cnt_hi, vlb_lo, vlb_hi, cnt_max, True)
    nc_max = _xlane_max(jnp.maximum(nc_lo, nc_hi))[0]

    kmin_f = plsc.bitcast(jnp.full((_L,), _INT_MIN, jnp.int32), jnp.float32)

    def clear_body(s, _):
        plsc.store_scatter(buf, [(s * _F) + iota], kmin_f, mask=s >= nc_lo)
        plsc.store_scatter(buf, [(s * _F) + iota + _L], kmin_f, mask=s >= nc_hi)
        return 0

    lax.fori_loop(0, nc_max, clear_body, 0)

    def bit_body(i, t):
        t_lo, t_hi = t
        bit = jnp.int32(1) << (jnp.int32(31) - i)
        cand_lo = t_lo + bit
        cand_hi = t_hi + bit

        def s_body(s, c):
            k_lo = plsc.bitcast(slot_lo(s), jnp.int32)
            k_hi = plsc.bitcast(slot_hi(s), jnp.int32)
            return (c[0] + jnp.where(k_lo >= cand_lo, 1, 0),
                    c[1] + jnp.where(k_hi >= cand_hi, 1, 0))

        c_lo, c_hi = lax.fori_loop(0, nc_max, s_body, (zi, zi))
        return (jnp.where(c_lo >= _K, cand_lo, t_lo),
                jnp.where(c_hi >= _K, cand_hi, t_hi))

    tmin = jnp.full((_L,), _INT_MIN, jnp.int32)
    t_lo, t_hi = lax.fori_loop(0, 32, bit_body, (tmin, tmin))

    def stat_body(s, c):
        sg_lo, sg_hi, cg_lo, cg_hi = c
        k_lo = plsc.bitcast(slot_lo(s), jnp.int32)
        k_hi = plsc.bitcast(slot_hi(s), jnp.int32)
        g_lo = k_lo > t_lo
        g_hi = k_hi > t_hi
        return (sg_lo + jnp.where(g_lo, _val_of(k_lo), 0.0),
                sg_hi + jnp.where(g_hi, _val_of(k_hi), 0.0),
                cg_lo + jnp.where(g_lo, 1, 0),
                cg_hi + jnp.where(g_hi, 1, 0))

    zf = jnp.zeros((_L,), jnp.float32)
    sg_lo, sg_hi, cg_lo, cg_hi = lax.fori_loop(0, nc_max, stat_body,
                                               (zf, zf, zi, zi))

    tie_lo = jnp.where(t_lo == _INT_MIN, 0.0, _val_of(t_lo))
    tie_hi = jnp.where(t_hi == _INT_MIN, 0.0, _val_of(t_hi))
    kf = jnp.float32(_K)
    denom = jnp.maximum(jnp.minimum(kf, _xlane_sum(nval).astype(jnp.float32)), 1.0)
    res_lo = (sg_lo + (kf - cg_lo.astype(jnp.float32)) * tie_lo) / denom
    res_hi = (sg_hi + (kf - cg_hi.astype(jnp.float32)) * tie_hi) / denom
    ov[pl.ds(0, _L)] = res_lo
    ov[pl.ds(_L, _L)] = res_hi
    pltpu.sync_copy(ov, out_hbm.at[w])


@jax.jit
def _sc_call(x, pen):
    mesh = plsc.VectorSubcoreMesh(core_axis_name="c", subcore_axis_name="s")
    return pl.kernel(
        _sc_body,
        mesh=mesh,
        compiler_params=pltpu.CompilerParams(needs_layout_passes=False),
        out_type=jax.ShapeDtypeStruct((_B, _F), jnp.float32),
        scratch_types=[
            pltpu.VMEM((_CHUNK, _F), jnp.float32),
            pltpu.VMEM((_CHUNK, _F), jnp.float32),
            pltpu.VMEM((_CHUNK,), jnp.float32),
            pltpu.VMEM((_CHUNK,), jnp.float32),
            pltpu.VMEM((_CAP * _F,), jnp.float32),
            pltpu.VMEM((_F,), jnp.float32),
            pltpu.SemaphoreType.DMA,
            pltpu.SemaphoreType.DMA,
            pltpu.SemaphoreType.DMA,
            pltpu.SemaphoreType.DMA,
        ],
    )(x, pen)


def kernel(x, mask):
    b, n, f = x.shape
    assert (b, n, f) == (_B, _N, _F)
    pen = jnp.where(mask, -jnp.inf, jnp.inf).astype(jnp.float32)
    return _sc_call(x, pen)
